# baseline (speedup 1.0000x reference)
import functools

import jax
import jax.numpy as jnp
from jax import lax
from jax.experimental import pallas as pl
from jax.experimental.pallas import tpu as pltpu
from jax.experimental.pallas import tpu_sc as plsc

_NW = 32
_SC_CHUNK = 32
_UNROLL = 8


def kernel(inputs, embeddings):
    B, S, D = inputs.shape
    rows_per_w = S // _NW
    n_seq_chunks = rows_per_w // _SC_CHUNK
    n_steps = n_seq_chunks * B
    CW = _SC_CHUNK * D

    mesh = plsc.VectorSubcoreMesh(core_axis_name="c", subcore_axis_name="s")

    @functools.partial(
        pl.kernel,
        mesh=mesh,
        out_type=jax.ShapeDtypeStruct((B * S * D,), jnp.float32),
        scratch_types=[
            pltpu.VMEM((2, CW), jnp.float32),
            pltpu.VMEM((CW,), jnp.float32),
            pltpu.SemaphoreType.DMA((2,)),
        ],
    )
    def sc_add(x_hbm, e_hbm, o_hbm, xb, eb, sout):
        wid = lax.axis_index("s") * 2 + lax.axis_index("c")
        srow = wid * rows_per_w

        def x_off(t):
            b = t % B
            q = t // B
            return (b * S + srow + q * _SC_CHUNK) * D

        for t in range(n_steps):
            cur = t % 2
            if t >= 2:
                pltpu.make_async_copy(
                    xb.at[cur],
                    o_hbm.at[pl.ds(x_off(t - 2), CW)],
                    sout.at[cur],
                ).wait()
            pltpu.sync_copy(x_hbm.at[pl.ds(x_off(t), CW)], xb.at[cur])
            if t % B == 0:
                pltpu.sync_copy(
                    e_hbm.at[pl.ds((srow + (t // B) * _SC_CHUNK) * D, CW)], eb
                )

            def add_body(i, c, cur=cur):
                base = i * (16 * _UNROLL)
                for u in range(_UNROLL):
                    o = base + u * 16
                    xb[cur, pl.ds(o, 16)] = (
                        xb[cur, pl.ds(o, 16)] + eb[pl.ds(o, 16)]
                    )
                return c

            lax.fori_loop(0, CW // (16 * _UNROLL), add_body, 0)
            pltpu.async_copy(
                xb.at[cur], o_hbm.at[pl.ds(x_off(t), CW)], sout.at[cur]
            )
        for t in (n_steps - 2, n_steps - 1):
            pltpu.make_async_copy(
                xb.at[t % 2], o_hbm.at[pl.ds(x_off(t), CW)], sout.at[t % 2]
            ).wait()

    out = sc_add(inputs.reshape(B * S * D), embeddings.reshape(S * D))
    return out.reshape(B, S, D)


# speedup vs baseline: 1.4814x; 1.4814x over previous
import functools

import jax
import jax.numpy as jnp
from jax import lax
from jax.experimental import pallas as pl
from jax.experimental.pallas import tpu as pltpu
from jax.experimental.pallas import tpu_sc as plsc

_NW = 32
_SC_CHUNK = 32
_UNROLL = 16


def kernel(inputs, embeddings):
    B, S, D = inputs.shape
    rows_per_w = S // _NW
    n_chunks = rows_per_w // _SC_CHUNK
    CW = _SC_CHUNK * D

    mesh = plsc.VectorSubcoreMesh(core_axis_name="c", subcore_axis_name="s")

    @functools.partial(
        pl.kernel,
        mesh=mesh,
        out_type=jax.ShapeDtypeStruct((B * S * D,), jnp.float32),
        scratch_types=[
            pltpu.VMEM((CW,), jnp.float32),
            pltpu.VMEM((CW,), jnp.float32),
        ],
    )
    def sc_add(x_hbm, e_hbm, o_hbm, xb, eb):
        wid = lax.axis_index("s") * 2 + lax.axis_index("c")
        srow = wid * rows_per_w

        def chunk_body(t, carry):
            row0 = srow + t * _SC_CHUNK
            pltpu.sync_copy(e_hbm.at[pl.ds(row0 * D, CW)], eb)
            for b in range(B):
                x_off = (b * S + row0) * D
                pltpu.sync_copy(x_hbm.at[pl.ds(x_off, CW)], xb)

                def add_body(i, c):
                    base = i * (16 * _UNROLL)
                    for u in range(_UNROLL):
                        o = base + u * 16
                        xb[pl.ds(o, 16)] = xb[pl.ds(o, 16)] + eb[pl.ds(o, 16)]
                    return c

                lax.fori_loop(0, CW // (16 * _UNROLL), add_body, 0)
                pltpu.sync_copy(xb, o_hbm.at[pl.ds(x_off, CW)])
            return carry

        lax.fori_loop(0, n_chunks, chunk_body, 0)

    out = sc_add(inputs.reshape(B * S * D), embeddings.reshape(S * D))
    return out.reshape(B, S, D)
